# baseline (speedup 1.0000x reference)
import jax
import jax.numpy as jnp
from jax import lax
from jax.experimental import pallas as pl
from jax.experimental.pallas import tpu as pltpu
from jax.experimental.pallas import tpu_sc as plsc

_N = 10000
_E = 320000
_D = 128
_T = 1000
_NC = 2
_NS = 16
_NW = _NC * _NS
_EPW = _E // _NW
_KE = 40
_ECH = _EPW // _KE
_IB = 50
_NIB = _ECH // _IB
_NPAD = 10240
_NPW = _NPAD // _NW
_KN = 40
_NNCH = _NPW // _KN
_NACC = 10240
_RPT = _NACC // _NS
_OCH = _RPT // _KE


def _sc_gather_body(types_ref, emb1_ref, h1_ref, idxv, gbuf, sem):
    cid = lax.axis_index("c")
    sid = lax.axis_index("s")
    wid = sid * _NC + cid

    pltpu.sync_copy(types_ref.at[wid], idxv)
    for j in range(_NNCH):
        pltpu.async_copy(emb1_ref.at[idxv.at[j]], gbuf, sem).wait()
        pltpu.sync_copy(gbuf, h1_ref.at[pl.ds(wid * _NPW + j * _KN, _KN)])


def _sc_scatter_body(g_ref, row_ref, col_ref, zeros_ref,
                     sa_ref, sb_ref,
                     rowv, colv, gbuf0, gbuf1, acc, sem0, sem1):
    cid = lax.axis_index("c")
    sid = lax.axis_index("s")
    wid = sid * _NC + cid

    pltpu.sync_copy(zeros_ref, gbuf0)
    for t in range(_OCH):
        pltpu.sync_copy(gbuf0, acc.at[pl.ds(sid * _RPT + t * _KE, _KE)])
    plsc.subcore_barrier()

    def _block(b, carry):
        pltpu.sync_copy(row_ref.at[wid * _NIB + b], rowv)
        pltpu.sync_copy(col_ref.at[wid * _NIB + b], colv)

        def _chunk(j, c2):
            pltpu.async_copy(g_ref.at[rowv.at[j]], gbuf0, sem0).wait()
            pltpu.sync_copy(gbuf0, acc.at[colv.at[j]], add=True)
            return c2

        lax.fori_loop(0, _IB, _chunk, 0)
        return carry

    lax.fori_loop(0, _NIB, _block, 0)
    plsc.subcore_barrier()

    for t in range(_OCH):
        pltpu.sync_copy(acc.at[pl.ds(sid * _RPT + t * _KE, _KE)], gbuf0)

        @pl.when(cid == 0)
        def _():
            pltpu.sync_copy(gbuf0,
                            sa_ref.at[pl.ds(sid * _RPT + t * _KE, _KE)])

        @pl.when(cid == 1)
        def _():
            pltpu.sync_copy(gbuf0,
                            sb_ref.at[pl.ds(sid * _RPT + t * _KE, _KE)])


def _tc_mm_body(a_ref, w_ref, o_ref):
    o_ref[...] = jnp.dot(a_ref[...], w_ref[...],
                         preferred_element_type=jnp.float32)


def _tc_norm_body(da_ref, db_ref, h1_ref, g1_ref, dinvb_ref):
    dinvb = lax.rsqrt(da_ref[...] + db_ref[...] + 1.0)
    dinvb_ref[...] = dinvb
    g1_ref[...] = dinvb * h1_ref[...]


def _tc_layer_body(g1_ref, sa_ref, sb_ref, dinvb_ref, b1_ref, w2_ref,
                   g2_ref):
    dinvb = dinvb_ref[...]
    y1 = jnp.maximum(
        dinvb * (sa_ref[...] + sb_ref[...] + g1_ref[...]) + b1_ref[...], 0.0)
    h2 = jnp.dot(y1, w2_ref[...], preferred_element_type=jnp.float32)
    g2_ref[...] = dinvb * h2


def _tc_out_body(g2_ref, sa_ref, sb_ref, dinvb_ref, b2_ref, o_ref):
    o_ref[...] = (dinvb_ref[...]
                  * (sa_ref[...] + sb_ref[...] + g2_ref[...]) + b2_ref[...])


_f32 = jnp.float32
_R = 1000


def _sc_mesh():
    return plsc.VectorSubcoreMesh(core_axis_name="c", subcore_axis_name="s")


def _gather_call(types_p, emb1):
    return pl.kernel(
        _sc_gather_body,
        out_type=jax.ShapeDtypeStruct((_NPAD, _D), _f32),
        mesh=_sc_mesh(),
        scratch_types=[
            pltpu.VMEM((_NNCH, _KN), jnp.int32),
            pltpu.VMEM((_KN, _D), _f32),
            pltpu.SemaphoreType.DMA,
        ],
    )(types_p, emb1)


def _scatter_call(g, row3, col3, zeros128):
    return pl.kernel(
        _sc_scatter_body,
        out_type=(
            jax.ShapeDtypeStruct((_NACC, _D), _f32),
            jax.ShapeDtypeStruct((_NACC, _D), _f32),
        ),
        mesh=_sc_mesh(),
        scratch_types=[
            pltpu.VMEM((_IB, _KE), jnp.int32),
            pltpu.VMEM((_IB, _KE), jnp.int32),
            pltpu.VMEM((_KE, _D), _f32),
            pltpu.VMEM((_KE, _D), _f32),
            pltpu.VMEM_SHARED((_NACC, _D), _f32),
            pltpu.SemaphoreType.DMA,
            pltpu.SemaphoreType.DMA,
        ],
    )(g, row3, col3, zeros128)


def kernel(x_node_types, edge_index, emb_table, W1, b1, W2, b2):
    types = x_node_types.astype(jnp.int32)
    row3 = edge_index[0].astype(jnp.int32).reshape(_NW * _NIB, _IB, _KE)
    col3 = edge_index[1].astype(jnp.int32).reshape(_NW * _NIB, _IB, _KE)
    types_p = jnp.pad(types, (0, _NPAD - _N)).reshape(_NW, _NNCH, _KN)
    ones_h = jnp.ones((_N, _D), _f32)
    zeros128 = jnp.zeros((_KE, _D), _f32)
    b1r = b1.reshape(1, _D).astype(_f32)
    b2r = b2.reshape(1, _D).astype(_f32)

    emb1 = pl.pallas_call(
        _tc_mm_body,
        out_shape=jax.ShapeDtypeStruct((_T, _D), _f32),
    )(emb_table.astype(_f32), W1.astype(_f32))

    h1 = _gather_call(types_p, emb1)

    row3z = jnp.zeros_like(row3)
    da, db = _scatter_call(ones_h, row3z, col3, zeros128)

    grid = (_N // _R,)
    g1, dinvb = pl.pallas_call(
        _tc_norm_body,
        grid=grid,
        in_specs=[
            pl.BlockSpec((_R, _D), lambda i: (i, 0)),
            pl.BlockSpec((_R, _D), lambda i: (i, 0)),
            pl.BlockSpec((_R, _D), lambda i: (i, 0)),
        ],
        out_specs=[
            pl.BlockSpec((_R, _D), lambda i: (i, 0)),
            pl.BlockSpec((_R, _D), lambda i: (i, 0)),
        ],
        out_shape=[
            jax.ShapeDtypeStruct((_N, _D), _f32),
            jax.ShapeDtypeStruct((_N, _D), _f32),
        ],
    )(da, db, h1)

    s1a, s1b = _scatter_call(g1, row3, col3, zeros128)

    g2 = pl.pallas_call(
        _tc_layer_body,
        grid=grid,
        in_specs=[
            pl.BlockSpec((_R, _D), lambda i: (i, 0)),
            pl.BlockSpec((_R, _D), lambda i: (i, 0)),
            pl.BlockSpec((_R, _D), lambda i: (i, 0)),
            pl.BlockSpec((_R, _D), lambda i: (i, 0)),
            pl.BlockSpec((1, _D), lambda i: (0, 0)),
            pl.BlockSpec((_D, _D), lambda i: (0, 0)),
        ],
        out_specs=pl.BlockSpec((_R, _D), lambda i: (i, 0)),
        out_shape=jax.ShapeDtypeStruct((_N, _D), _f32),
    )(g1, s1a, s1b, dinvb, b1r, W2.astype(_f32))

    s2a, s2b = _scatter_call(g2, row3, col3, zeros128)

    out = pl.pallas_call(
        _tc_out_body,
        grid=grid,
        in_specs=[
            pl.BlockSpec((_R, _D), lambda i: (i, 0)),
            pl.BlockSpec((_R, _D), lambda i: (i, 0)),
            pl.BlockSpec((_R, _D), lambda i: (i, 0)),
            pl.BlockSpec((_R, _D), lambda i: (i, 0)),
            pl.BlockSpec((1, _D), lambda i: (0, 0)),
        ],
        out_specs=pl.BlockSpec((_R, _D), lambda i: (i, 0)),
        out_shape=jax.ShapeDtypeStruct((_N, _D), _f32),
    )(g2, s2a, s2b, dinvb, b2r)
    return out


# speedup vs baseline: 13.1531x; 13.1531x over previous
import jax
import jax.numpy as jnp
from jax import lax
from jax.experimental import pallas as pl
from jax.experimental.pallas import tpu as pltpu
from jax.experimental.pallas import tpu_sc as plsc

_N = 10000
_E = 320000
_D = 128
_T = 1000
_NC = 2
_NS = 16
_NW = _NC * _NS
_EPW = _E // _NW
_KE = 40
_ECH = _EPW // _KE
_IB = 50
_NIB = _ECH // _IB
_NPAD = 10240
_NPW = _NPAD // _NW
_KN = 40
_NNCH = _NPW // _KN
_NACC = 10240
_RPT = _NACC // _NS
_OCH = _RPT // _KE


def _sc_gather_body(types_ref, emb1_ref, h1_ref, idxv, gbuf, sem):
    cid = lax.axis_index("c")
    sid = lax.axis_index("s")
    wid = sid * _NC + cid

    pltpu.sync_copy(types_ref.at[wid], idxv)
    for j in range(_NNCH):
        pltpu.async_copy(emb1_ref.at[idxv.at[j]], gbuf, sem).wait()
        pltpu.sync_copy(gbuf, h1_ref.at[pl.ds(wid * _NPW + j * _KN, _KN)])


def _sc_scatter_body(g_ref, row_ref, col_ref, zeros_ref,
                     sa_ref, sb_ref,
                     rowv, colv, gbuf0, gbuf1, acc, sem0, sem1):
    cid = lax.axis_index("c")
    sid = lax.axis_index("s")
    wid = sid * _NC + cid

    pltpu.sync_copy(zeros_ref, gbuf0)
    for t in range(_OCH):
        pltpu.sync_copy(gbuf0, acc.at[pl.ds(sid * _RPT + t * _KE, _KE)])
    plsc.subcore_barrier()

    def _block(b, carry):
        pltpu.sync_copy(row_ref.at[wid * _NIB + b], rowv)
        pltpu.sync_copy(col_ref.at[wid * _NIB + b], colv)

        def _chunk(j, c2):
            pltpu.async_copy(g_ref.at[rowv.at[j]], gbuf0, sem0).wait()
            pltpu.sync_copy(gbuf0, acc.at[colv.at[j]], add=True)
            return c2

        lax.fori_loop(0, _IB, _chunk, 0)
        return carry

    lax.fori_loop(0, _NIB, _block, 0)
    plsc.subcore_barrier()

    for t in range(_OCH):
        pltpu.sync_copy(acc.at[pl.ds(sid * _RPT + t * _KE, _KE)], gbuf0)

        @pl.when(cid == 0)
        def _():
            pltpu.sync_copy(gbuf0,
                            sa_ref.at[pl.ds(sid * _RPT + t * _KE, _KE)])

        @pl.when(cid == 1)
        def _():
            pltpu.sync_copy(gbuf0,
                            sb_ref.at[pl.ds(sid * _RPT + t * _KE, _KE)])


def _tc_mm_body(a_ref, w_ref, o_ref):
    o_ref[...] = jnp.dot(a_ref[...], w_ref[...],
                         preferred_element_type=jnp.float32)


def _tc_norm_body(da_ref, db_ref, h1_ref, g1_ref, dinvb_ref):
    dinvb = lax.rsqrt(da_ref[...] + db_ref[...] + 1.0)
    dinvb_ref[...] = dinvb
    g1_ref[...] = dinvb * h1_ref[...]


def _tc_layer_body(g1_ref, sa_ref, sb_ref, dinvb_ref, b1_ref, w2_ref,
                   g2_ref):
    dinvb = dinvb_ref[...]
    y1 = jnp.maximum(
        dinvb * (sa_ref[...] + sb_ref[...] + g1_ref[...]) + b1_ref[...], 0.0)
    h2 = jnp.dot(y1, w2_ref[...], preferred_element_type=jnp.float32)
    g2_ref[...] = dinvb * h2


def _tc_out_body(g2_ref, sa_ref, sb_ref, dinvb_ref, b2_ref, o_ref):
    o_ref[...] = (dinvb_ref[...]
                  * (sa_ref[...] + sb_ref[...] + g2_ref[...]) + b2_ref[...])


_f32 = jnp.float32
_R = 1000


def _sc_mesh():
    return plsc.VectorSubcoreMesh(core_axis_name="c", subcore_axis_name="s")


def _gather_call(types_p, emb1):
    return pl.kernel(
        _sc_gather_body,
        out_type=jax.ShapeDtypeStruct((_NPAD, _D), _f32),
        mesh=_sc_mesh(),
        scratch_types=[
            pltpu.VMEM((_NNCH, _KN), jnp.int32),
            pltpu.VMEM((_KN, _D), _f32),
            pltpu.SemaphoreType.DMA,
        ],
    )(types_p, emb1)


def _scatter_call(g, row3, col3, zeros128):
    return pl.kernel(
        _sc_scatter_body,
        out_type=(
            jax.ShapeDtypeStruct((_NACC, _D), _f32),
            jax.ShapeDtypeStruct((_NACC, _D), _f32),
        ),
        mesh=_sc_mesh(),
        scratch_types=[
            pltpu.VMEM((_IB, _KE), jnp.int32),
            pltpu.VMEM((_IB, _KE), jnp.int32),
            pltpu.VMEM((_KE, _D), _f32),
            pltpu.VMEM((_KE, _D), _f32),
            pltpu.VMEM_SHARED((_NACC, _D), _f32),
            pltpu.SemaphoreType.DMA,
            pltpu.SemaphoreType.DMA,
        ],
    )(g, row3, col3, zeros128)


def kernel(x_node_types, edge_index, emb_table, W1, b1, W2, b2):
    types = x_node_types.astype(jnp.int32)
    row3 = edge_index[0].astype(jnp.int32).reshape(_NW * _NIB, _IB, _KE)
    col3 = edge_index[1].astype(jnp.int32).reshape(_NW * _NIB, _IB, _KE)
    types_p = jnp.pad(types, (0, _NPAD - _N)).reshape(_NW, _NNCH, _KN)
    ones_h = jnp.ones((_N, _D), _f32)
    zeros128 = jnp.zeros((_KE, _D), _f32)
    b1r = b1.reshape(1, _D).astype(_f32)
    b2r = b2.reshape(1, _D).astype(_f32)

    emb1 = pl.pallas_call(
        _tc_mm_body,
        out_shape=jax.ShapeDtypeStruct((_T, _D), _f32),
    )(emb_table.astype(_f32), W1.astype(_f32))

    h1 = _gather_call(types_p, emb1)

    da, db = _scatter_call(ones_h, row3, col3, zeros128)

    grid = (_N // _R,)
    g1, dinvb = pl.pallas_call(
        _tc_norm_body,
        grid=grid,
        in_specs=[
            pl.BlockSpec((_R, _D), lambda i: (i, 0)),
            pl.BlockSpec((_R, _D), lambda i: (i, 0)),
            pl.BlockSpec((_R, _D), lambda i: (i, 0)),
        ],
        out_specs=[
            pl.BlockSpec((_R, _D), lambda i: (i, 0)),
            pl.BlockSpec((_R, _D), lambda i: (i, 0)),
        ],
        out_shape=[
            jax.ShapeDtypeStruct((_N, _D), _f32),
            jax.ShapeDtypeStruct((_N, _D), _f32),
        ],
    )(da, db, h1)

    s1a, s1b = _scatter_call(g1, row3, col3, zeros128)

    g2 = pl.pallas_call(
        _tc_layer_body,
        grid=grid,
        in_specs=[
            pl.BlockSpec((_R, _D), lambda i: (i, 0)),
            pl.BlockSpec((_R, _D), lambda i: (i, 0)),
            pl.BlockSpec((_R, _D), lambda i: (i, 0)),
            pl.BlockSpec((_R, _D), lambda i: (i, 0)),
            pl.BlockSpec((1, _D), lambda i: (0, 0)),
            pl.BlockSpec((_D, _D), lambda i: (0, 0)),
        ],
        out_specs=pl.BlockSpec((_R, _D), lambda i: (i, 0)),
        out_shape=jax.ShapeDtypeStruct((_N, _D), _f32),
    )(g1, s1a, s1b, dinvb, b1r, W2.astype(_f32))

    s2a, s2b = _scatter_call(g2, row3, col3, zeros128)

    out = pl.pallas_call(
        _tc_out_body,
        grid=grid,
        in_specs=[
            pl.BlockSpec((_R, _D), lambda i: (i, 0)),
            pl.BlockSpec((_R, _D), lambda i: (i, 0)),
            pl.BlockSpec((_R, _D), lambda i: (i, 0)),
            pl.BlockSpec((_R, _D), lambda i: (i, 0)),
            pl.BlockSpec((1, _D), lambda i: (0, 0)),
        ],
        out_specs=pl.BlockSpec((_R, _D), lambda i: (i, 0)),
        out_shape=jax.ShapeDtypeStruct((_N, _D), _f32),
    )(g2, s2a, s2b, dinvb, b2r)
    return out


# speedup vs baseline: 18.4259x; 1.4009x over previous
import jax
import jax.numpy as jnp
from jax import lax
from jax.experimental import pallas as pl
from jax.experimental.pallas import tpu as pltpu
from jax.experimental.pallas import tpu_sc as plsc

_N = 10000
_E = 320000
_D = 128
_T = 1000
_NC = 2
_NS = 16
_NW = _NC * _NS
_EPW = _E // _NW
_KE = 40
_ECH = _EPW // _KE
_IB = 50
_NIB = _ECH // _IB
_NPAD = 10240
_NPW = _NPAD // _NW
_KN = 40
_NNCH = _NPW // _KN
_NACC = 10240
_RPT = _NACC // _NS
_OCH = _RPT // _KE


def _sc_gather_body(types_ref, emb1_ref, h1_ref, idxv, gbuf, sem):
    cid = lax.axis_index("c")
    sid = lax.axis_index("s")
    wid = sid * _NC + cid

    pltpu.sync_copy(types_ref.at[wid], idxv)
    for j in range(_NNCH):
        pltpu.async_copy(emb1_ref.at[idxv.at[j]], gbuf, sem).wait()
        pltpu.sync_copy(gbuf, h1_ref.at[pl.ds(wid * _NPW + j * _KN, _KN)])


def _sc_scatter_body(g_ref, row_ref, col_ref, zeros_ref,
                     sa_ref, sb_ref,
                     rowv, colv, gbuf0, gbuf1, acc, sem0, sem1):
    cid = lax.axis_index("c")
    sid = lax.axis_index("s")
    wid = sid * _NC + cid

    pltpu.sync_copy(zeros_ref, gbuf0)
    for t in range(_OCH):
        pltpu.sync_copy(gbuf0, acc.at[pl.ds(sid * _RPT + t * _KE, _KE)])
    plsc.subcore_barrier()

    def _block(b, carry):
        pltpu.sync_copy(row_ref.at[wid * _NIB + b], rowv)
        pltpu.sync_copy(col_ref.at[wid * _NIB + b], colv)

        def _pair(p, c2):
            j0 = 2 * p
            cp0 = pltpu.async_copy(g_ref.at[rowv.at[j0]], gbuf0, sem0)
            cp1 = pltpu.async_copy(g_ref.at[rowv.at[j0 + 1]], gbuf1, sem1)
            cp0.wait()
            pltpu.sync_copy(gbuf0, acc.at[colv.at[j0]], add=True)
            cp1.wait()
            pltpu.sync_copy(gbuf1, acc.at[colv.at[j0 + 1]], add=True)
            return c2

        lax.fori_loop(0, _IB // 2, _pair, 0)
        return carry

    lax.fori_loop(0, _NIB, _block, 0)
    plsc.subcore_barrier()

    for t in range(_OCH):
        pltpu.sync_copy(acc.at[pl.ds(sid * _RPT + t * _KE, _KE)], gbuf0)

        @pl.when(cid == 0)
        def _():
            pltpu.sync_copy(gbuf0,
                            sa_ref.at[pl.ds(sid * _RPT + t * _KE, _KE)])

        @pl.when(cid == 1)
        def _():
            pltpu.sync_copy(gbuf0,
                            sb_ref.at[pl.ds(sid * _RPT + t * _KE, _KE)])


def _tc_mm_body(a_ref, w_ref, o_ref):
    o_ref[...] = jnp.dot(a_ref[...], w_ref[...],
                         preferred_element_type=jnp.float32)


def _tc_norm_body(da_ref, db_ref, h1_ref, g1_ref, dinvb_ref):
    dinvb = lax.rsqrt(da_ref[...] + db_ref[...] + 1.0)
    dinvb_ref[...] = dinvb
    g1_ref[...] = dinvb * h1_ref[...]


def _tc_layer_body(g1_ref, sa_ref, sb_ref, dinvb_ref, b1_ref, w2_ref,
                   g2_ref):
    dinvb = dinvb_ref[...]
    y1 = jnp.maximum(
        dinvb * (sa_ref[...] + sb_ref[...] + g1_ref[...]) + b1_ref[...], 0.0)
    h2 = jnp.dot(y1, w2_ref[...], preferred_element_type=jnp.float32)
    g2_ref[...] = dinvb * h2


def _tc_out_body(g2_ref, sa_ref, sb_ref, dinvb_ref, b2_ref, o_ref):
    o_ref[...] = (dinvb_ref[...]
                  * (sa_ref[...] + sb_ref[...] + g2_ref[...]) + b2_ref[...])


_f32 = jnp.float32
_R = 1000


def _sc_mesh():
    return plsc.VectorSubcoreMesh(core_axis_name="c", subcore_axis_name="s")


def _gather_call(types_p, emb1):
    return pl.kernel(
        _sc_gather_body,
        out_type=jax.ShapeDtypeStruct((_NPAD, _D), _f32),
        mesh=_sc_mesh(),
        scratch_types=[
            pltpu.VMEM((_NNCH, _KN), jnp.int32),
            pltpu.VMEM((_KN, _D), _f32),
            pltpu.SemaphoreType.DMA,
        ],
    )(types_p, emb1)


def _scatter_call(g, row3, col3, zeros128):
    return pl.kernel(
        _sc_scatter_body,
        out_type=(
            jax.ShapeDtypeStruct((_NACC, _D), _f32),
            jax.ShapeDtypeStruct((_NACC, _D), _f32),
        ),
        mesh=_sc_mesh(),
        scratch_types=[
            pltpu.VMEM((_IB, _KE), jnp.int32),
            pltpu.VMEM((_IB, _KE), jnp.int32),
            pltpu.VMEM((_KE, _D), _f32),
            pltpu.VMEM((_KE, _D), _f32),
            pltpu.VMEM_SHARED((_NACC, _D), _f32),
            pltpu.SemaphoreType.DMA,
            pltpu.SemaphoreType.DMA,
        ],
    )(g, row3, col3, zeros128)


def kernel(x_node_types, edge_index, emb_table, W1, b1, W2, b2):
    types = x_node_types.astype(jnp.int32)
    row3 = edge_index[0].astype(jnp.int32).reshape(_NW * _NIB, _IB, _KE)
    col3 = edge_index[1].astype(jnp.int32).reshape(_NW * _NIB, _IB, _KE)
    types_p = jnp.pad(types, (0, _NPAD - _N)).reshape(_NW, _NNCH, _KN)
    ones_h = jnp.ones((_N, _D), _f32)
    zeros128 = jnp.zeros((_KE, _D), _f32)
    b1r = b1.reshape(1, _D).astype(_f32)
    b2r = b2.reshape(1, _D).astype(_f32)

    emb1 = pl.pallas_call(
        _tc_mm_body,
        out_shape=jax.ShapeDtypeStruct((_T, _D), _f32),
    )(emb_table.astype(_f32), W1.astype(_f32))

    h1 = _gather_call(types_p, emb1)

    da, db = _scatter_call(ones_h, row3, col3, zeros128)

    grid = (_N // _R,)
    g1, dinvb = pl.pallas_call(
        _tc_norm_body,
        grid=grid,
        in_specs=[
            pl.BlockSpec((_R, _D), lambda i: (i, 0)),
            pl.BlockSpec((_R, _D), lambda i: (i, 0)),
            pl.BlockSpec((_R, _D), lambda i: (i, 0)),
        ],
        out_specs=[
            pl.BlockSpec((_R, _D), lambda i: (i, 0)),
            pl.BlockSpec((_R, _D), lambda i: (i, 0)),
        ],
        out_shape=[
            jax.ShapeDtypeStruct((_N, _D), _f32),
            jax.ShapeDtypeStruct((_N, _D), _f32),
        ],
    )(da, db, h1)

    s1a, s1b = _scatter_call(g1, row3, col3, zeros128)

    g2 = pl.pallas_call(
        _tc_layer_body,
        grid=grid,
        in_specs=[
            pl.BlockSpec((_R, _D), lambda i: (i, 0)),
            pl.BlockSpec((_R, _D), lambda i: (i, 0)),
            pl.BlockSpec((_R, _D), lambda i: (i, 0)),
            pl.BlockSpec((_R, _D), lambda i: (i, 0)),
            pl.BlockSpec((1, _D), lambda i: (0, 0)),
            pl.BlockSpec((_D, _D), lambda i: (0, 0)),
        ],
        out_specs=pl.BlockSpec((_R, _D), lambda i: (i, 0)),
        out_shape=jax.ShapeDtypeStruct((_N, _D), _f32),
    )(g1, s1a, s1b, dinvb, b1r, W2.astype(_f32))

    s2a, s2b = _scatter_call(g2, row3, col3, zeros128)

    out = pl.pallas_call(
        _tc_out_body,
        grid=grid,
        in_specs=[
            pl.BlockSpec((_R, _D), lambda i: (i, 0)),
            pl.BlockSpec((_R, _D), lambda i: (i, 0)),
            pl.BlockSpec((_R, _D), lambda i: (i, 0)),
            pl.BlockSpec((_R, _D), lambda i: (i, 0)),
            pl.BlockSpec((1, _D), lambda i: (0, 0)),
        ],
        out_specs=pl.BlockSpec((_R, _D), lambda i: (i, 0)),
        out_shape=jax.ShapeDtypeStruct((_N, _D), _f32),
    )(g2, s2a, s2b, dinvb, b2r)
    return out


# speedup vs baseline: 21.7432x; 1.1800x over previous
import jax
import jax.numpy as jnp
from jax import lax
from jax.experimental import pallas as pl
from jax.experimental.pallas import tpu as pltpu
from jax.experimental.pallas import tpu_sc as plsc

_N = 10000
_E = 320000
_D = 128
_T = 1000
_NC = 2
_NS = 16
_NW = _NC * _NS
_EPW = _E // _NW
_KE = 40
_ECH = _EPW // _KE
_IB = 50
_NIB = _ECH // _IB
_NPAD = 10240
_NPW = _NPAD // _NW
_KN = 40
_NNCH = _NPW // _KN
_NACC = 10240
_RPT = _NACC // _NS
_OCH = _RPT // _KE


def _sc_gather_body(types_ref, emb1_ref, h1_ref, idxv, gbuf, sem):
    cid = lax.axis_index("c")
    sid = lax.axis_index("s")
    wid = sid * _NC + cid

    pltpu.sync_copy(types_ref.at[wid], idxv)
    for j in range(_NNCH):
        pltpu.async_copy(emb1_ref.at[idxv.at[j]], gbuf, sem).wait()
        pltpu.sync_copy(gbuf, h1_ref.at[pl.ds(wid * _NPW + j * _KN, _KN)])


def _sc_scatter_body(g_ref, row_ref, col_ref, zeros_ref,
                     sa_ref, sb_ref,
                     rowv, colv, gbuf0, gbuf1, acc, sem0, sem1):
    cid = lax.axis_index("c")
    sid = lax.axis_index("s")
    wid = sid * _NC + cid

    pltpu.sync_copy(zeros_ref, gbuf0)
    for t in range(_OCH):
        pltpu.sync_copy(gbuf0, acc.at[pl.ds(sid * _RPT + t * _KE, _KE)])
    plsc.subcore_barrier()

    def _block(b, carry):
        pltpu.sync_copy(row_ref.at[wid * _NIB + b], rowv)
        pltpu.sync_copy(col_ref.at[wid * _NIB + b], colv)

        pltpu.async_copy(g_ref.at[rowv.at[0]], gbuf0, sem0)
        pltpu.async_copy(g_ref.at[rowv.at[1]], gbuf1, sem1)

        def _pair(p, c2):
            j0 = 2 * p
            pltpu.make_async_copy(g_ref.at[rowv.at[j0]], gbuf0, sem0).wait()
            pltpu.sync_copy(gbuf0, acc.at[colv.at[j0]], add=True)

            @pl.when(j0 + 2 < _IB)
            def _():
                pltpu.async_copy(g_ref.at[rowv.at[j0 + 2]], gbuf0, sem0)

            pltpu.make_async_copy(g_ref.at[rowv.at[j0 + 1]], gbuf1,
                                  sem1).wait()
            pltpu.sync_copy(gbuf1, acc.at[colv.at[j0 + 1]], add=True)

            @pl.when(j0 + 3 < _IB)
            def _():
                pltpu.async_copy(g_ref.at[rowv.at[j0 + 3]], gbuf1, sem1)

            return c2

        lax.fori_loop(0, _IB // 2, _pair, 0)
        return carry

    lax.fori_loop(0, _NIB, _block, 0)
    plsc.subcore_barrier()

    for t in range(_OCH):
        pltpu.sync_copy(acc.at[pl.ds(sid * _RPT + t * _KE, _KE)], gbuf0)

        @pl.when(cid == 0)
        def _():
            pltpu.sync_copy(gbuf0,
                            sa_ref.at[pl.ds(sid * _RPT + t * _KE, _KE)])

        @pl.when(cid == 1)
        def _():
            pltpu.sync_copy(gbuf0,
                            sb_ref.at[pl.ds(sid * _RPT + t * _KE, _KE)])


def _tc_mm_body(a_ref, w_ref, o_ref):
    o_ref[...] = jnp.dot(a_ref[...], w_ref[...],
                         preferred_element_type=jnp.float32)


def _tc_norm_body(da_ref, db_ref, h1_ref, g1_ref, dinvb_ref):
    dinvb = lax.rsqrt(da_ref[...] + db_ref[...] + 1.0)
    dinvb_ref[...] = dinvb
    g1_ref[...] = dinvb * h1_ref[...]


def _tc_layer_body(g1_ref, sa_ref, sb_ref, dinvb_ref, b1_ref, w2_ref,
                   g2_ref):
    dinvb = dinvb_ref[...]
    y1 = jnp.maximum(
        dinvb * (sa_ref[...] + sb_ref[...] + g1_ref[...]) + b1_ref[...], 0.0)
    h2 = jnp.dot(y1, w2_ref[...], preferred_element_type=jnp.float32)
    g2_ref[...] = dinvb * h2


def _tc_out_body(g2_ref, sa_ref, sb_ref, dinvb_ref, b2_ref, o_ref):
    o_ref[...] = (dinvb_ref[...]
                  * (sa_ref[...] + sb_ref[...] + g2_ref[...]) + b2_ref[...])


_f32 = jnp.float32
_R = 1000


def _sc_mesh():
    return plsc.VectorSubcoreMesh(core_axis_name="c", subcore_axis_name="s")


def _gather_call(types_p, emb1):
    return pl.kernel(
        _sc_gather_body,
        out_type=jax.ShapeDtypeStruct((_NPAD, _D), _f32),
        mesh=_sc_mesh(),
        scratch_types=[
            pltpu.VMEM((_NNCH, _KN), jnp.int32),
            pltpu.VMEM((_KN, _D), _f32),
            pltpu.SemaphoreType.DMA,
        ],
    )(types_p, emb1)


def _scatter_call(g, row3, col3, zeros128):
    return pl.kernel(
        _sc_scatter_body,
        out_type=(
            jax.ShapeDtypeStruct((_NACC, _D), _f32),
            jax.ShapeDtypeStruct((_NACC, _D), _f32),
        ),
        mesh=_sc_mesh(),
        scratch_types=[
            pltpu.VMEM((_IB, _KE), jnp.int32),
            pltpu.VMEM((_IB, _KE), jnp.int32),
            pltpu.VMEM((_KE, _D), _f32),
            pltpu.VMEM((_KE, _D), _f32),
            pltpu.VMEM_SHARED((_NACC, _D), _f32),
            pltpu.SemaphoreType.DMA,
            pltpu.SemaphoreType.DMA,
        ],
    )(g, row3, col3, zeros128)


def kernel(x_node_types, edge_index, emb_table, W1, b1, W2, b2):
    types = x_node_types.astype(jnp.int32)
    row3 = edge_index[0].astype(jnp.int32).reshape(_NW * _NIB, _IB, _KE)
    col3 = edge_index[1].astype(jnp.int32).reshape(_NW * _NIB, _IB, _KE)
    types_p = jnp.pad(types, (0, _NPAD - _N)).reshape(_NW, _NNCH, _KN)
    ones_h = jnp.ones((_N, _D), _f32)
    zeros128 = jnp.zeros((_KE, _D), _f32)
    b1r = b1.reshape(1, _D).astype(_f32)
    b2r = b2.reshape(1, _D).astype(_f32)

    emb1 = pl.pallas_call(
        _tc_mm_body,
        out_shape=jax.ShapeDtypeStruct((_T, _D), _f32),
    )(emb_table.astype(_f32), W1.astype(_f32))

    h1 = _gather_call(types_p, emb1)

    da, db = _scatter_call(ones_h, row3, col3, zeros128)

    grid = (_N // _R,)
    g1, dinvb = pl.pallas_call(
        _tc_norm_body,
        grid=grid,
        in_specs=[
            pl.BlockSpec((_R, _D), lambda i: (i, 0)),
            pl.BlockSpec((_R, _D), lambda i: (i, 0)),
            pl.BlockSpec((_R, _D), lambda i: (i, 0)),
        ],
        out_specs=[
            pl.BlockSpec((_R, _D), lambda i: (i, 0)),
            pl.BlockSpec((_R, _D), lambda i: (i, 0)),
        ],
        out_shape=[
            jax.ShapeDtypeStruct((_N, _D), _f32),
            jax.ShapeDtypeStruct((_N, _D), _f32),
        ],
    )(da, db, h1)

    s1a, s1b = _scatter_call(g1, row3, col3, zeros128)

    g2 = pl.pallas_call(
        _tc_layer_body,
        grid=grid,
        in_specs=[
            pl.BlockSpec((_R, _D), lambda i: (i, 0)),
            pl.BlockSpec((_R, _D), lambda i: (i, 0)),
            pl.BlockSpec((_R, _D), lambda i: (i, 0)),
            pl.BlockSpec((_R, _D), lambda i: (i, 0)),
            pl.BlockSpec((1, _D), lambda i: (0, 0)),
            pl.BlockSpec((_D, _D), lambda i: (0, 0)),
        ],
        out_specs=pl.BlockSpec((_R, _D), lambda i: (i, 0)),
        out_shape=jax.ShapeDtypeStruct((_N, _D), _f32),
    )(g1, s1a, s1b, dinvb, b1r, W2.astype(_f32))

    s2a, s2b = _scatter_call(g2, row3, col3, zeros128)

    out = pl.pallas_call(
        _tc_out_body,
        grid=grid,
        in_specs=[
            pl.BlockSpec((_R, _D), lambda i: (i, 0)),
            pl.BlockSpec((_R, _D), lambda i: (i, 0)),
            pl.BlockSpec((_R, _D), lambda i: (i, 0)),
            pl.BlockSpec((_R, _D), lambda i: (i, 0)),
            pl.BlockSpec((1, _D), lambda i: (0, 0)),
        ],
        out_specs=pl.BlockSpec((_R, _D), lambda i: (i, 0)),
        out_shape=jax.ShapeDtypeStruct((_N, _D), _f32),
    )(g2, s2a, s2b, dinvb, b2r)
    return out


# speedup vs baseline: 30.4869x; 1.4021x over previous
import jax
import jax.numpy as jnp
from jax import lax
from jax.experimental import pallas as pl
from jax.experimental.pallas import tpu as pltpu
from jax.experimental.pallas import tpu_sc as plsc

_N = 10000
_E = 320000
_D = 128
_T = 1000
_NC = 2
_NS = 16
_NW = _NC * _NS
_EPW = _E // _NW
_KE = 40
_ECH = _EPW // _KE
_IB = 50
_NIB = _ECH // _IB
_NPAD = 10240
_NPW = _NPAD // _NW
_KN = 40
_NNCH = _NPW // _KN
_NACC = 10240
_RPT = _NACC // _NS
_OCH = _RPT // _KE


def _sc_gather_body(types_ref, emb1_ref, h1_ref, idxv, gbuf, sem):
    cid = lax.axis_index("c")
    sid = lax.axis_index("s")
    wid = sid * _NC + cid

    pltpu.sync_copy(types_ref.at[wid], idxv)
    for j in range(_NNCH):
        pltpu.async_copy(emb1_ref.at[idxv.at[j]], gbuf, sem).wait()
        pltpu.sync_copy(gbuf, h1_ref.at[pl.ds(wid * _NPW + j * _KN, _KN)])


def _sc_scatter_body(g_ref, row_ref, col_ref, zeros_ref,
                     sa_ref, sb_ref,
                     rowv, colv, gbuf0, gbuf1, gbuf2, gbuf3, acc,
                     sem0, sem1, sem2, sem3):
    cid = lax.axis_index("c")
    sid = lax.axis_index("s")
    wid = sid * _NC + cid

    pltpu.sync_copy(zeros_ref, gbuf0)
    for t in range(_OCH):
        pltpu.sync_copy(gbuf0, acc.at[pl.ds(sid * _RPT + t * _KE, _KE)])
    plsc.subcore_barrier()

    def _block(b, carry):
        pltpu.sync_copy(row_ref.at[wid * _NIB + b], rowv)
        pltpu.sync_copy(col_ref.at[wid * _NIB + b], colv)

        bufs = (gbuf0, gbuf1, gbuf2, gbuf3)
        sems = (sem0, sem1, sem2, sem3)
        for k in range(4):
            pltpu.async_copy(g_ref.at[rowv.at[k]], bufs[k], sems[k])

        def _quad(q, c2):
            j0 = 4 * q
            for k in range(4):
                pltpu.make_async_copy(g_ref.at[rowv.at[j0 + k]], bufs[k],
                                      sems[k]).wait()
                pltpu.sync_copy(bufs[k], acc.at[colv.at[j0 + k]], add=True)

                @pl.when(j0 + k + 4 < _IB)
                def _():
                    pltpu.async_copy(g_ref.at[rowv.at[j0 + k + 4]], bufs[k],
                                     sems[k])

            return c2

        lax.fori_loop(0, _IB // 4, _quad, 0)

        for k in range(_IB - (_IB // 4) * 4):
            j = (_IB // 4) * 4 + k
            pltpu.make_async_copy(g_ref.at[rowv.at[j]], bufs[k],
                                  sems[k]).wait()
            pltpu.sync_copy(bufs[k], acc.at[colv.at[j]], add=True)
        return carry

    lax.fori_loop(0, _NIB, _block, 0)
    plsc.subcore_barrier()

    for t in range(_OCH):
        pltpu.sync_copy(acc.at[pl.ds(sid * _RPT + t * _KE, _KE)], gbuf0)

        @pl.when(cid == 0)
        def _():
            pltpu.sync_copy(gbuf0,
                            sa_ref.at[pl.ds(sid * _RPT + t * _KE, _KE)])

        @pl.when(cid == 1)
        def _():
            pltpu.sync_copy(gbuf0,
                            sb_ref.at[pl.ds(sid * _RPT + t * _KE, _KE)])


def _tc_mm_body(a_ref, w_ref, o_ref):
    o_ref[...] = jnp.dot(a_ref[...], w_ref[...],
                         preferred_element_type=jnp.float32)


def _tc_norm_body(da_ref, db_ref, h1_ref, g1_ref, dinvb_ref):
    dinvb = lax.rsqrt(da_ref[...] + db_ref[...] + 1.0)
    dinvb_ref[...] = dinvb
    g1_ref[...] = dinvb * h1_ref[...]


def _tc_layer_body(g1_ref, sa_ref, sb_ref, dinvb_ref, b1_ref, w2_ref,
                   g2_ref):
    dinvb = dinvb_ref[...]
    y1 = jnp.maximum(
        dinvb * (sa_ref[...] + sb_ref[...] + g1_ref[...]) + b1_ref[...], 0.0)
    h2 = jnp.dot(y1, w2_ref[...], preferred_element_type=jnp.float32)
    g2_ref[...] = dinvb * h2


def _tc_out_body(g2_ref, sa_ref, sb_ref, dinvb_ref, b2_ref, o_ref):
    o_ref[...] = (dinvb_ref[...]
                  * (sa_ref[...] + sb_ref[...] + g2_ref[...]) + b2_ref[...])


_f32 = jnp.float32
_R = 1000


def _sc_mesh():
    return plsc.VectorSubcoreMesh(core_axis_name="c", subcore_axis_name="s")


def _gather_call(types_p, emb1):
    return pl.kernel(
        _sc_gather_body,
        out_type=jax.ShapeDtypeStruct((_NPAD, _D), _f32),
        mesh=_sc_mesh(),
        scratch_types=[
            pltpu.VMEM((_NNCH, _KN), jnp.int32),
            pltpu.VMEM((_KN, _D), _f32),
            pltpu.SemaphoreType.DMA,
        ],
    )(types_p, emb1)


def _scatter_call(g, row3, col3, zeros128):
    return pl.kernel(
        _sc_scatter_body,
        out_type=(
            jax.ShapeDtypeStruct((_NACC, _D), _f32),
            jax.ShapeDtypeStruct((_NACC, _D), _f32),
        ),
        mesh=_sc_mesh(),
        scratch_types=[
            pltpu.VMEM((_IB, _KE), jnp.int32),
            pltpu.VMEM((_IB, _KE), jnp.int32),
            pltpu.VMEM((_KE, _D), _f32),
            pltpu.VMEM((_KE, _D), _f32),
            pltpu.VMEM((_KE, _D), _f32),
            pltpu.VMEM((_KE, _D), _f32),
            pltpu.VMEM_SHARED((_NACC, _D), _f32),
            pltpu.SemaphoreType.DMA,
            pltpu.SemaphoreType.DMA,
            pltpu.SemaphoreType.DMA,
            pltpu.SemaphoreType.DMA,
        ],
    )(g, row3, col3, zeros128)


def kernel(x_node_types, edge_index, emb_table, W1, b1, W2, b2):
    types = x_node_types.astype(jnp.int32)
    row3 = edge_index[0].astype(jnp.int32).reshape(_NW * _NIB, _IB, _KE)
    col3 = edge_index[1].astype(jnp.int32).reshape(_NW * _NIB, _IB, _KE)
    types_p = jnp.pad(types, (0, _NPAD - _N)).reshape(_NW, _NNCH, _KN)
    ones_h = jnp.ones((_N, _D), _f32)
    zeros128 = jnp.zeros((_KE, _D), _f32)
    b1r = b1.reshape(1, _D).astype(_f32)
    b2r = b2.reshape(1, _D).astype(_f32)

    emb1 = pl.pallas_call(
        _tc_mm_body,
        out_shape=jax.ShapeDtypeStruct((_T, _D), _f32),
    )(emb_table.astype(_f32), W1.astype(_f32))

    h1 = _gather_call(types_p, emb1)

    da, db = _scatter_call(ones_h, row3, col3, zeros128)

    grid = (_N // _R,)
    g1, dinvb = pl.pallas_call(
        _tc_norm_body,
        grid=grid,
        in_specs=[
            pl.BlockSpec((_R, _D), lambda i: (i, 0)),
            pl.BlockSpec((_R, _D), lambda i: (i, 0)),
            pl.BlockSpec((_R, _D), lambda i: (i, 0)),
        ],
        out_specs=[
            pl.BlockSpec((_R, _D), lambda i: (i, 0)),
            pl.BlockSpec((_R, _D), lambda i: (i, 0)),
        ],
        out_shape=[
            jax.ShapeDtypeStruct((_N, _D), _f32),
            jax.ShapeDtypeStruct((_N, _D), _f32),
        ],
    )(da, db, h1)

    s1a, s1b = _scatter_call(g1, row3, col3, zeros128)

    g2 = pl.pallas_call(
        _tc_layer_body,
        grid=grid,
        in_specs=[
            pl.BlockSpec((_R, _D), lambda i: (i, 0)),
            pl.BlockSpec((_R, _D), lambda i: (i, 0)),
            pl.BlockSpec((_R, _D), lambda i: (i, 0)),
            pl.BlockSpec((_R, _D), lambda i: (i, 0)),
            pl.BlockSpec((1, _D), lambda i: (0, 0)),
            pl.BlockSpec((_D, _D), lambda i: (0, 0)),
        ],
        out_specs=pl.BlockSpec((_R, _D), lambda i: (i, 0)),
        out_shape=jax.ShapeDtypeStruct((_N, _D), _f32),
    )(g1, s1a, s1b, dinvb, b1r, W2.astype(_f32))

    s2a, s2b = _scatter_call(g2, row3, col3, zeros128)

    out = pl.pallas_call(
        _tc_out_body,
        grid=grid,
        in_specs=[
            pl.BlockSpec((_R, _D), lambda i: (i, 0)),
            pl.BlockSpec((_R, _D), lambda i: (i, 0)),
            pl.BlockSpec((_R, _D), lambda i: (i, 0)),
            pl.BlockSpec((_R, _D), lambda i: (i, 0)),
            pl.BlockSpec((1, _D), lambda i: (0, 0)),
        ],
        out_specs=pl.BlockSpec((_R, _D), lambda i: (i, 0)),
        out_shape=jax.ShapeDtypeStruct((_N, _D), _f32),
    )(g2, s2a, s2b, dinvb, b2r)
    return out


# speedup vs baseline: 31.3391x; 1.0280x over previous
import jax
import jax.numpy as jnp
from jax import lax
from jax.experimental import pallas as pl
from jax.experimental.pallas import tpu as pltpu
from jax.experimental.pallas import tpu_sc as plsc

_N = 10000
_E = 320000
_D = 128
_T = 1000
_NC = 2
_NS = 16
_NW = _NC * _NS
_EPW = _E // _NW
_KE = 40
_ECH = _EPW // _KE
_IB = 50
_NIB = _ECH // _IB
_NPAD = 10240
_NPW = _NPAD // _NW
_KN = 40
_NNCH = _NPW // _KN
_NACC = 10240
_RPT = _NACC // _NS
_OCH = _RPT // _KE


def _sc_gather_body(types_ref, emb1_ref, h1_ref, idxv, gbuf, sem):
    cid = lax.axis_index("c")
    sid = lax.axis_index("s")
    wid = sid * _NC + cid

    pltpu.sync_copy(types_ref.at[wid], idxv)
    for j in range(_NNCH):
        pltpu.async_copy(emb1_ref.at[idxv.at[j]], gbuf, sem).wait()
        pltpu.sync_copy(gbuf, h1_ref.at[pl.ds(wid * _NPW + j * _KN, _KN)])


def _sc_scatter_body(g_ref, row_ref, col_ref, zeros_ref,
                     sa_ref, sb_ref,
                     rowv, colv, gbuf0, gbuf1, gbuf2, gbuf3, gbuf4, gbuf5,
                     acc, sem0, sem1, sem2, sem3, sem4, sem5):
    cid = lax.axis_index("c")
    sid = lax.axis_index("s")
    wid = sid * _NC + cid

    pltpu.sync_copy(zeros_ref, gbuf0)
    for t in range(_OCH):
        pltpu.sync_copy(gbuf0, acc.at[pl.ds(sid * _RPT + t * _KE, _KE)])
    plsc.subcore_barrier()

    def _block(b, carry):
        pltpu.sync_copy(row_ref.at[wid * _NIB + b], rowv)
        pltpu.sync_copy(col_ref.at[wid * _NIB + b], colv)

        bufs = (gbuf0, gbuf1, gbuf2, gbuf3, gbuf4, gbuf5)
        sems = (sem0, sem1, sem2, sem3, sem4, sem5)
        nd = 6
        for k in range(nd):
            pltpu.async_copy(g_ref.at[rowv.at[k]], bufs[k], sems[k])

        def _group(q, c2):
            j0 = nd * q
            for k in range(nd):
                pltpu.make_async_copy(g_ref.at[rowv.at[j0 + k]], bufs[k],
                                      sems[k]).wait()
                pltpu.sync_copy(bufs[k], acc.at[colv.at[j0 + k]], add=True)

                @pl.when(j0 + k + nd < _IB)
                def _():
                    pltpu.async_copy(g_ref.at[rowv.at[j0 + k + nd]], bufs[k],
                                     sems[k])

            return c2

        lax.fori_loop(0, _IB // nd, _group, 0)

        for k in range(_IB - (_IB // nd) * nd):
            j = (_IB // nd) * nd + k
            pltpu.make_async_copy(g_ref.at[rowv.at[j]], bufs[k],
                                  sems[k]).wait()
            pltpu.sync_copy(bufs[k], acc.at[colv.at[j]], add=True)
        return carry

    lax.fori_loop(0, _NIB, _block, 0)
    plsc.subcore_barrier()

    for t in range(_OCH):
        pltpu.sync_copy(acc.at[pl.ds(sid * _RPT + t * _KE, _KE)], gbuf0)

        @pl.when(cid == 0)
        def _():
            pltpu.sync_copy(gbuf0,
                            sa_ref.at[pl.ds(sid * _RPT + t * _KE, _KE)])

        @pl.when(cid == 1)
        def _():
            pltpu.sync_copy(gbuf0,
                            sb_ref.at[pl.ds(sid * _RPT + t * _KE, _KE)])


def _tc_mm_body(a_ref, w_ref, o_ref):
    o_ref[...] = jnp.dot(a_ref[...], w_ref[...],
                         preferred_element_type=jnp.float32)


def _tc_norm_body(da_ref, db_ref, h1_ref, g1_ref, dinvb_ref):
    dinvb = lax.rsqrt(da_ref[...] + db_ref[...] + 1.0)
    dinvb_ref[...] = dinvb
    g1_ref[...] = dinvb * h1_ref[...]


def _tc_layer_body(g1_ref, sa_ref, sb_ref, dinvb_ref, b1_ref, w2_ref,
                   g2_ref):
    dinvb = dinvb_ref[...]
    y1 = jnp.maximum(
        dinvb * (sa_ref[...] + sb_ref[...] + g1_ref[...]) + b1_ref[...], 0.0)
    h2 = jnp.dot(y1, w2_ref[...], preferred_element_type=jnp.float32)
    g2_ref[...] = dinvb * h2


def _tc_out_body(g2_ref, sa_ref, sb_ref, dinvb_ref, b2_ref, o_ref):
    o_ref[...] = (dinvb_ref[...]
                  * (sa_ref[...] + sb_ref[...] + g2_ref[...]) + b2_ref[...])


_f32 = jnp.float32
_R = 1000


def _sc_mesh():
    return plsc.VectorSubcoreMesh(core_axis_name="c", subcore_axis_name="s")


def _gather_call(types_p, emb1):
    return pl.kernel(
        _sc_gather_body,
        out_type=jax.ShapeDtypeStruct((_NPAD, _D), _f32),
        mesh=_sc_mesh(),
        scratch_types=[
            pltpu.VMEM((_NNCH, _KN), jnp.int32),
            pltpu.VMEM((_KN, _D), _f32),
            pltpu.SemaphoreType.DMA,
        ],
    )(types_p, emb1)


def _scatter_call(g, row3, col3, zeros128):
    return pl.kernel(
        _sc_scatter_body,
        out_type=(
            jax.ShapeDtypeStruct((_NACC, _D), _f32),
            jax.ShapeDtypeStruct((_NACC, _D), _f32),
        ),
        mesh=_sc_mesh(),
        scratch_types=[
            pltpu.VMEM((_IB, _KE), jnp.int32),
            pltpu.VMEM((_IB, _KE), jnp.int32),
            pltpu.VMEM((_KE, _D), _f32),
            pltpu.VMEM((_KE, _D), _f32),
            pltpu.VMEM((_KE, _D), _f32),
            pltpu.VMEM((_KE, _D), _f32),
            pltpu.VMEM((_KE, _D), _f32),
            pltpu.VMEM((_KE, _D), _f32),
            pltpu.VMEM_SHARED((_NACC, _D), _f32),
            pltpu.SemaphoreType.DMA,
            pltpu.SemaphoreType.DMA,
            pltpu.SemaphoreType.DMA,
            pltpu.SemaphoreType.DMA,
            pltpu.SemaphoreType.DMA,
            pltpu.SemaphoreType.DMA,
        ],
    )(g, row3, col3, zeros128)


def kernel(x_node_types, edge_index, emb_table, W1, b1, W2, b2):
    types = x_node_types.astype(jnp.int32)
    row3 = edge_index[0].astype(jnp.int32).reshape(_NW * _NIB, _IB, _KE)
    col3 = edge_index[1].astype(jnp.int32).reshape(_NW * _NIB, _IB, _KE)
    types_p = jnp.pad(types, (0, _NPAD - _N)).reshape(_NW, _NNCH, _KN)
    ones_h = jnp.ones((_N, _D), _f32)
    zeros128 = jnp.zeros((_KE, _D), _f32)
    b1r = b1.reshape(1, _D).astype(_f32)
    b2r = b2.reshape(1, _D).astype(_f32)

    emb1 = pl.pallas_call(
        _tc_mm_body,
        out_shape=jax.ShapeDtypeStruct((_T, _D), _f32),
    )(emb_table.astype(_f32), W1.astype(_f32))

    h1 = _gather_call(types_p, emb1)

    da, db = _scatter_call(ones_h, row3, col3, zeros128)

    grid = (_N // _R,)
    g1, dinvb = pl.pallas_call(
        _tc_norm_body,
        grid=grid,
        in_specs=[
            pl.BlockSpec((_R, _D), lambda i: (i, 0)),
            pl.BlockSpec((_R, _D), lambda i: (i, 0)),
            pl.BlockSpec((_R, _D), lambda i: (i, 0)),
        ],
        out_specs=[
            pl.BlockSpec((_R, _D), lambda i: (i, 0)),
            pl.BlockSpec((_R, _D), lambda i: (i, 0)),
        ],
        out_shape=[
            jax.ShapeDtypeStruct((_N, _D), _f32),
            jax.ShapeDtypeStruct((_N, _D), _f32),
        ],
    )(da, db, h1)

    s1a, s1b = _scatter_call(g1, row3, col3, zeros128)

    g2 = pl.pallas_call(
        _tc_layer_body,
        grid=grid,
        in_specs=[
            pl.BlockSpec((_R, _D), lambda i: (i, 0)),
            pl.BlockSpec((_R, _D), lambda i: (i, 0)),
            pl.BlockSpec((_R, _D), lambda i: (i, 0)),
            pl.BlockSpec((_R, _D), lambda i: (i, 0)),
            pl.BlockSpec((1, _D), lambda i: (0, 0)),
            pl.BlockSpec((_D, _D), lambda i: (0, 0)),
        ],
        out_specs=pl.BlockSpec((_R, _D), lambda i: (i, 0)),
        out_shape=jax.ShapeDtypeStruct((_N, _D), _f32),
    )(g1, s1a, s1b, dinvb, b1r, W2.astype(_f32))

    s2a, s2b = _scatter_call(g2, row3, col3, zeros128)

    out = pl.pallas_call(
        _tc_out_body,
        grid=grid,
        in_specs=[
            pl.BlockSpec((_R, _D), lambda i: (i, 0)),
            pl.BlockSpec((_R, _D), lambda i: (i, 0)),
            pl.BlockSpec((_R, _D), lambda i: (i, 0)),
            pl.BlockSpec((_R, _D), lambda i: (i, 0)),
            pl.BlockSpec((1, _D), lambda i: (0, 0)),
        ],
        out_specs=pl.BlockSpec((_R, _D), lambda i: (i, 0)),
        out_shape=jax.ShapeDtypeStruct((_N, _D), _f32),
    )(g2, s2a, s2b, dinvb, b2r)
    return out
